# baseline (speedup 1.0000x reference)
import numpy as np
import jax
import jax.numpy as jnp
from jax import lax
from jax.experimental import pallas as pl
from jax.experimental.pallas import tpu as pltpu
from jax.experimental.pallas import tpu_sc as plsc

_T = 6
_NP = 16
_NN = 32
_NE = 16
_P96 = _T * _NP
_K = 6


def _build_counts():
    C = np.zeros((_T, _NP, _P96), np.float32)
    for i in range(_T):
        Ci = i + 1

        def tf(cp, npp):
            m = cp * 16 + npp
            return (m % Ci) * 16 + (m // Ci)

        for j in range(_NP):
            C[i, j, tf(Ci - 1, j)] += 1.0
        il = 2
        for t in range(i, -1, -1):
            for j in range(_NP):
                for k in range(-il + 1, il):
                    for l in range(-il + 1, il):
                        idx = j + 16 * k + l
                        if (not (j == 0 and l == 0 and il == 2)) and 0 <= idx < _NP:
                            C[i, j, tf(t, idx)] += 1.0
            il += 1
    return C.reshape(_P96, _P96)


_C_PAIR = _build_counts()


def _iota():
    return lax.iota(jnp.int32, _NE)


def _take(v, idx):
    return jnp.take_along_axis(v, idx, axis=0, mode="promise_in_bounds")


def _bc(v, lane):
    return _take(v, jnp.full((_NE,), lane, jnp.int32))


def _allsum(v):
    for sh in (1, 2, 4, 8):
        v = v + _take(v, _iota() ^ sh)
    return v


def _tree(vs):
    while len(vs) > 1:
        vs = [vs[a] + vs[a + 1] for a in range(0, len(vs) - 1, 2)] + (
            [vs[-1]] if len(vs) % 2 else [])
    return vs[0]


def _swap_step(rows, rb, lb):
    s = 1 << rb
    L = 1 << lb
    lanesel = (_iota() & L) == 0
    perm = _iota() ^ L
    out = list(rows)
    for i in range(16):
        if i & s:
            continue
        a_, b_ = rows[i], rows[i | s]
        out[i] = jnp.where(lanesel, a_, _take(b_, perm))
        out[i | s] = jnp.where(lanesel, _take(a_, perm), b_)
    return out


def _transpose16(rows):
    for s in range(4):
        rows = _swap_step(rows, s, s)
    return rows


def _sc_body(x_hbm, wq_hbm, wk_hbm, bk_hbm, wv_hbm, bv_hbm, c_hbm, out_hbm,
             x_v, wq_v, wk_v, bk_v, wv_v, bv_v, c_v, p_v, pt_v, wqt_v, wvt_v,
             out_v, sem):
    wid = lax.axis_index("s")
    jj = [lax.rem(wid * _K + k, _NP) for k in range(_K)]
    copies = [
        pltpu.async_copy(x_hbm, x_v, sem),
        pltpu.async_copy(wq_hbm, wq_v, sem),
        pltpu.async_copy(bk_hbm, bk_v, sem),
        pltpu.async_copy(bv_hbm, bv_v, sem),
        pltpu.async_copy(c_hbm.at[wid], c_v, sem),
    ]
    for k in range(_K):
        copies.append(pltpu.async_copy(wk_hbm.at[jj[k]], wk_v.at[k], sem))
        copies.append(pltpu.async_copy(wv_hbm.at[jj[k]], wv_v.at[k], sem))
    for c in copies:
        c.wait()

    zeros = jnp.zeros((_NE,), jnp.float32)

    for t in range(_T):
        rows = [x_v[t * 16 + r] for r in range(16)]
        rows = _swap_step(rows, 0, 2)
        rows = _swap_step(rows, 1, 3)
        for n in range(16):
            p_v[t * 16 + n] = rows[n]
        rows = _transpose16(rows)
        for e in range(16):
            pt_v[t * 16 + e] = rows[e]

    for blk in range(2):
        rows = _transpose16([wq_v[blk * 16 + c] for c in range(16)])
        for e in range(16):
            wqt_v[blk * 16 + e] = rows[e]

    for k in range(_K):
        for blk in range(2):
            rows = _transpose16([wv_v[k, blk * 16 + c] for c in range(16)])
            for e in range(16):
                wvt_v[k, blk * 16 + e] = rows[e]

    for g in range(2):
        ks = [g * 3 + kk for kk in range(3)]

        pqk = []
        for k in ks:
            m = wid * _K + k
            qm = lax.rem(m, _T) * 16 + lax.div(m, _T)
            pqk.append(p_v[qm])
        q0 = [zeros] * 3
        q1 = [zeros] * 3
        for e in range(_NE):
            r0 = wqt_v[e]
            r1 = wqt_v[16 + e]
            for ki in range(3):
                pe = _bc(pqk[ki], e)
                q0[ki] = q0[ki] + pe * r0
                q1[ki] = q1[ki] + pe * r1

        a = [zeros] * 3
        beta = [None] * 3
        for ki, k in enumerate(ks):
            for c in range(_NN):
                qc = _bc(q0[ki] if c < 16 else q1[ki], c % 16)
                a[ki] = a[ki] + qc * wk_v[k, c]
            beta[ki] = _allsum(q0[ki] * bk_v[jj[k], 0:16]
                               + q1[ki] * bk_v[jj[k], 16:32])

        us = [[zeros] * _T for _ in range(3)]
        for e in range(_NE):
            ptr = [pt_v[ch * 16 + e] for ch in range(_T)]
            for ki in range(3):
                ae = _bc(a[ki], e)
                for ch in range(_T):
                    us[ki][ch] = us[ki][ch] + ae * ptr[ch]

        wch = [[None] * _T for _ in range(3)]
        for ki, k in enumerate(ks):
            for ch in range(_T):
                wch[ki][ch] = c_v[k, pl.ds(ch * 16, 16)] * (us[ki][ch] + beta[ki])
        tparts = [[] for _ in range(3)]
        for ch in range(_T):
            prods = [[] for _ in range(3)]
            for pp in range(16):
                prow = p_v[ch * 16 + pp]
                for ki in range(3):
                    prods[ki].append(_bc(wch[ki][ch], pp) * prow)
            for ki in range(3):
                tparts[ki].append(_tree(prods[ki]))
        tv = [None] * 3
        sw = [None] * 3
        for ki in range(3):
            tv[ki] = _tree(tparts[ki])
            wtot = wch[ki][0]
            for ch in range(1, _T):
                wtot = wtot + wch[ki][ch]
            sw[ki] = _allsum(wtot)

        for ki, k in enumerate(ks):
            o0 = sw[ki] * bv_v[jj[k], 0:16]
            o1 = sw[ki] * bv_v[jj[k], 16:32]
            for e in range(_NE):
                te = _bc(tv[ki], e)
                o0 = o0 + te * wvt_v[k, e]
                o1 = o1 + te * wvt_v[k, 16 + e]
            out_v[k, 0:16] = o0
            out_v[k, 16:32] = o1

    pltpu.sync_copy(out_v, out_hbm.at[wid])


def _make_sc_call():
    return pl.kernel(
        _sc_body,
        out_type=jax.ShapeDtypeStruct((16, _K, _NN), jnp.float32),
        mesh=plsc.VectorSubcoreMesh(core_axis_name="c", subcore_axis_name="s",
                                    num_cores=1, num_subcores=16),
        scratch_types=[
            pltpu.VMEM((_P96, _NE), jnp.float32),
            pltpu.VMEM((_NN, _NE), jnp.float32),
            pltpu.VMEM((_K, _NN, _NE), jnp.float32),
            pltpu.VMEM((_NP, _NN), jnp.float32),
            pltpu.VMEM((_K, _NN, _NE), jnp.float32),
            pltpu.VMEM((_NP, _NN), jnp.float32),
            pltpu.VMEM((_K, _P96), jnp.float32),
            pltpu.VMEM((_P96, _NE), jnp.float32),
            pltpu.VMEM((_P96, _NE), jnp.float32),
            pltpu.VMEM((_NN, _NE), jnp.float32),
            pltpu.VMEM((_K, _NN, _NE), jnp.float32),
            pltpu.VMEM((_K, _NN), jnp.float32),
            pltpu.SemaphoreType.DMA,
        ],
    )


def kernel(x, Wq, Wk, bk, Wv, bv):
    out = _make_sc_call()(
        x.reshape(_P96, _NE),
        Wq,
        Wk,
        bk,
        Wv,
        bv,
        jnp.asarray(_C_PAIR.reshape(16, _K, _P96)),
    )
    return out.reshape(_T, _NP, _NN)[None]


# speedup vs baseline: 1.0411x; 1.0411x over previous
import numpy as np
import jax
import jax.numpy as jnp
from jax import lax
from jax.experimental import pallas as pl
from jax.experimental.pallas import tpu as pltpu
from jax.experimental.pallas import tpu_sc as plsc

_T = 6
_NP = 16
_NN = 32
_NE = 16
_P96 = _T * _NP
_K = 3


def _build_counts():
    C = np.zeros((_T, _NP, _P96), np.float32)
    for i in range(_T):
        Ci = i + 1

        def tf(cp, npp):
            m = cp * 16 + npp
            return (m % Ci) * 16 + (m // Ci)

        for j in range(_NP):
            C[i, j, tf(Ci - 1, j)] += 1.0
        il = 2
        for t in range(i, -1, -1):
            for j in range(_NP):
                for k in range(-il + 1, il):
                    for l in range(-il + 1, il):
                        idx = j + 16 * k + l
                        if (not (j == 0 and l == 0 and il == 2)) and 0 <= idx < _NP:
                            C[i, j, tf(t, idx)] += 1.0
            il += 1
    return C.reshape(_P96, _P96)


_C_PAIR = _build_counts()


def _iota():
    return lax.iota(jnp.int32, _NE)


def _take(v, idx):
    return jnp.take_along_axis(v, idx, axis=0, mode="promise_in_bounds")


def _bc(v, lane):
    return _take(v, jnp.full((_NE,), lane, jnp.int32))


def _allsum(v):
    for sh in (1, 2, 4, 8):
        v = v + _take(v, _iota() ^ sh)
    return v


def _tree(vs):
    while len(vs) > 1:
        vs = [vs[a] + vs[a + 1] for a in range(0, len(vs) - 1, 2)] + (
            [vs[-1]] if len(vs) % 2 else [])
    return vs[0]


def _swap_step(rows, rb, lb):
    s = 1 << rb
    L = 1 << lb
    lanesel = (_iota() & L) == 0
    perm = _iota() ^ L
    out = list(rows)
    for i in range(16):
        if i & s:
            continue
        a_, b_ = rows[i], rows[i | s]
        out[i] = jnp.where(lanesel, a_, _take(b_, perm))
        out[i | s] = jnp.where(lanesel, _take(a_, perm), b_)
    return out


def _transpose16(rows):
    for s in range(4):
        rows = _swap_step(rows, s, s)
    return rows


def _sc_body(x_hbm, wq_hbm, wk_hbm, bk_hbm, wv_hbm, bv_hbm, c_hbm, out_hbm,
             x_v, wq_v, wk_v, bk_v, wv_v, bv_v, c_v, p_v, pt_v, wqt_v, wvt_v,
             out_v, sem):
    wid = lax.axis_index("s") * 2 + lax.axis_index("c")
    jj = [lax.rem(wid * _K + k, _NP) for k in range(_K)]
    copies = [
        pltpu.async_copy(x_hbm, x_v, sem),
        pltpu.async_copy(wq_hbm, wq_v, sem),
        pltpu.async_copy(bk_hbm, bk_v, sem),
        pltpu.async_copy(bv_hbm, bv_v, sem),
        pltpu.async_copy(c_hbm.at[wid], c_v, sem),
    ]
    for k in range(_K):
        copies.append(pltpu.async_copy(wk_hbm.at[jj[k]], wk_v.at[k], sem))
        copies.append(pltpu.async_copy(wv_hbm.at[jj[k]], wv_v.at[k], sem))
    for c in copies:
        c.wait()

    zeros = jnp.zeros((_NE,), jnp.float32)

    for t in range(_T):
        rows = [x_v[t * 16 + r] for r in range(16)]
        rows = _swap_step(rows, 0, 2)
        rows = _swap_step(rows, 1, 3)
        for n in range(16):
            p_v[t * 16 + n] = rows[n]
        rows = _transpose16(rows)
        for e in range(16):
            pt_v[t * 16 + e] = rows[e]

    for blk in range(2):
        rows = _transpose16([wq_v[blk * 16 + c] for c in range(16)])
        for e in range(16):
            wqt_v[blk * 16 + e] = rows[e]

    for k in range(_K):
        for blk in range(2):
            rows = _transpose16([wv_v[k, blk * 16 + c] for c in range(16)])
            for e in range(16):
                wvt_v[k, blk * 16 + e] = rows[e]

    pqk = []
    for k in range(_K):
        m = wid * _K + k
        qm = lax.rem(m, _T) * 16 + lax.div(m, _T)
        pqk.append(p_v[qm])
    q0 = [zeros] * _K
    q1 = [zeros] * _K
    for e in range(_NE):
        r0 = wqt_v[e]
        r1 = wqt_v[16 + e]
        for k in range(_K):
            pe = _bc(pqk[k], e)
            q0[k] = q0[k] + pe * r0
            q1[k] = q1[k] + pe * r1

    a = [zeros] * _K
    beta = [None] * _K
    for k in range(_K):
        for c in range(_NN):
            qc = _bc(q0[k] if c < 16 else q1[k], c % 16)
            a[k] = a[k] + qc * wk_v[k, c]
        beta[k] = _allsum(q0[k] * bk_v[jj[k], 0:16] + q1[k] * bk_v[jj[k], 16:32])

    us = [[zeros] * _T for _ in range(_K)]
    for e in range(_NE):
        ptr = [pt_v[ch * 16 + e] for ch in range(_T)]
        for k in range(_K):
            ae = _bc(a[k], e)
            for ch in range(_T):
                us[k][ch] = us[k][ch] + ae * ptr[ch]

    wch = [[None] * _T for _ in range(_K)]
    for k in range(_K):
        for ch in range(_T):
            wch[k][ch] = c_v[k, pl.ds(ch * 16, 16)] * (us[k][ch] + beta[k])
    tparts = [[] for _ in range(_K)]
    for ch in range(_T):
        prods = [[] for _ in range(_K)]
        for pp in range(16):
            prow = p_v[ch * 16 + pp]
            for k in range(_K):
                prods[k].append(_bc(wch[k][ch], pp) * prow)
        for k in range(_K):
            tparts[k].append(_tree(prods[k]))
    tv = [None] * _K
    sw = [None] * _K
    for k in range(_K):
        tv[k] = _tree(tparts[k])
        wtot = wch[k][0]
        for ch in range(1, _T):
            wtot = wtot + wch[k][ch]
        sw[k] = _allsum(wtot)

    for k in range(_K):
        o0 = sw[k] * bv_v[jj[k], 0:16]
        o1 = sw[k] * bv_v[jj[k], 16:32]
        for e in range(_NE):
            te = _bc(tv[k], e)
            o0 = o0 + te * wvt_v[k, e]
            o1 = o1 + te * wvt_v[k, 16 + e]
        out_v[k, 0:16] = o0
        out_v[k, 16:32] = o1

    pltpu.sync_copy(out_v, out_hbm.at[wid])


def _make_sc_call():
    return pl.kernel(
        _sc_body,
        out_type=jax.ShapeDtypeStruct((32, _K, _NN), jnp.float32),
        mesh=plsc.VectorSubcoreMesh(core_axis_name="c", subcore_axis_name="s",
                                    num_cores=2, num_subcores=16),
        scratch_types=[
            pltpu.VMEM((_P96, _NE), jnp.float32),
            pltpu.VMEM((_NN, _NE), jnp.float32),
            pltpu.VMEM((_K, _NN, _NE), jnp.float32),
            pltpu.VMEM((_NP, _NN), jnp.float32),
            pltpu.VMEM((_K, _NN, _NE), jnp.float32),
            pltpu.VMEM((_NP, _NN), jnp.float32),
            pltpu.VMEM((_K, _P96), jnp.float32),
            pltpu.VMEM((_P96, _NE), jnp.float32),
            pltpu.VMEM((_P96, _NE), jnp.float32),
            pltpu.VMEM((_NN, _NE), jnp.float32),
            pltpu.VMEM((_K, _NN, _NE), jnp.float32),
            pltpu.VMEM((_K, _NN), jnp.float32),
            pltpu.SemaphoreType.DMA,
        ],
    )


def kernel(x, Wq, Wk, bk, Wv, bv):
    out = _make_sc_call()(
        x.reshape(_P96, _NE),
        Wq,
        Wk,
        bk,
        Wv,
        bv,
        jnp.asarray(_C_PAIR.reshape(32, _K, _P96)),
    )
    return out.reshape(_T, _NP, _NN)[None]


# speedup vs baseline: 1.0607x; 1.0188x over previous
import numpy as np
import jax
import jax.numpy as jnp
from jax import lax
from jax.experimental import pallas as pl
from jax.experimental.pallas import tpu as pltpu
from jax.experimental.pallas import tpu_sc as plsc

_T = 6
_NP = 16
_NN = 32
_NE = 16
_P96 = _T * _NP
_K = 3


def _build_counts():
    C = np.zeros((_T, _NP, _P96), np.float32)
    for i in range(_T):
        Ci = i + 1

        def tf(cp, npp):
            m = cp * 16 + npp
            return (m % Ci) * 16 + (m // Ci)

        for j in range(_NP):
            C[i, j, tf(Ci - 1, j)] += 1.0
        il = 2
        for t in range(i, -1, -1):
            for j in range(_NP):
                for k in range(-il + 1, il):
                    for l in range(-il + 1, il):
                        idx = j + 16 * k + l
                        if (not (j == 0 and l == 0 and il == 2)) and 0 <= idx < _NP:
                            C[i, j, tf(t, idx)] += 1.0
            il += 1
    return C.reshape(_P96, _P96)


_C_PAIR = _build_counts()


def _iota():
    return lax.iota(jnp.int32, _NE)


def _take(v, idx):
    return jnp.take_along_axis(v, idx, axis=0, mode="promise_in_bounds")


def _bc(v, lane):
    return _take(v, jnp.full((_NE,), lane, jnp.int32))


def _allsum(v):
    for sh in (1, 2, 4, 8):
        v = v + _take(v, _iota() ^ sh)
    return v


def _tree(vs):
    while len(vs) > 1:
        vs = [vs[a] + vs[a + 1] for a in range(0, len(vs) - 1, 2)] + (
            [vs[-1]] if len(vs) % 2 else [])
    return vs[0]


def _swap_step(rows, rb, lb):
    s = 1 << rb
    L = 1 << lb
    lanesel = (_iota() & L) == 0
    perm = _iota() ^ L
    out = list(rows)
    for i in range(16):
        if i & s:
            continue
        a_, b_ = rows[i], rows[i | s]
        out[i] = jnp.where(lanesel, a_, _take(b_, perm))
        out[i | s] = jnp.where(lanesel, _take(a_, perm), b_)
    return out


def _transpose16(rows):
    for s in range(4):
        rows = _swap_step(rows, s, s)
    return rows


def _sc_body(x_hbm, wq_hbm, wk_hbm, bk_hbm, wv_hbm, bv_hbm, c_hbm, out_hbm,
             x_v, wq_v, wk_v, bk_v, wv_v, bv_v, c_v, p_v, pt_v, wqt_v, wvt_v,
             out_v, sem, semx):
    wid = lax.axis_index("s") * 2 + lax.axis_index("c")
    jj = [lax.rem(wid * _K + k, _NP) for k in range(_K)]
    xcopy = pltpu.async_copy(x_hbm, x_v, semx)
    copies = [
        pltpu.async_copy(wq_hbm, wq_v, sem),
        pltpu.async_copy(bk_hbm, bk_v, sem),
        pltpu.async_copy(bv_hbm, bv_v, sem),
        pltpu.async_copy(c_hbm.at[wid], c_v, sem),
    ]
    for k in range(_K):
        copies.append(pltpu.async_copy(wk_hbm.at[jj[k]], wk_v.at[k], sem))
        copies.append(pltpu.async_copy(wv_hbm.at[jj[k]], wv_v.at[k], sem))
    xcopy.wait()

    zeros = jnp.zeros((_NE,), jnp.float32)

    for t in range(_T):
        rows = [x_v[t * 16 + r] for r in range(16)]
        rows = _swap_step(rows, 0, 2)
        rows = _swap_step(rows, 1, 3)
        for n in range(16):
            p_v[t * 16 + n] = rows[n]
        rows = _transpose16(rows)
        for e in range(16):
            pt_v[t * 16 + e] = rows[e]

    for c in copies:
        c.wait()

    for blk in range(2):
        rows = _transpose16([wq_v[blk * 16 + c] for c in range(16)])
        for e in range(16):
            wqt_v[blk * 16 + e] = rows[e]

    for k in range(_K):
        for blk in range(2):
            rows = _transpose16([wv_v[k, blk * 16 + c] for c in range(16)])
            for e in range(16):
                wvt_v[k, blk * 16 + e] = rows[e]

    pqk = []
    for k in range(_K):
        m = wid * _K + k
        qm = lax.rem(m, _T) * 16 + lax.div(m, _T)
        pqk.append(p_v[qm])
    q0 = [zeros] * _K
    q1 = [zeros] * _K
    for e in range(_NE):
        r0 = wqt_v[e]
        r1 = wqt_v[16 + e]
        for k in range(_K):
            pe = _bc(pqk[k], e)
            q0[k] = q0[k] + pe * r0
            q1[k] = q1[k] + pe * r1

    a = [zeros] * _K
    beta = [None] * _K
    for k in range(_K):
        for c in range(_NN):
            qc = _bc(q0[k] if c < 16 else q1[k], c % 16)
            a[k] = a[k] + qc * wk_v[k, c]
        beta[k] = _allsum(q0[k] * bk_v[jj[k], 0:16] + q1[k] * bk_v[jj[k], 16:32])

    us = [[zeros] * _T for _ in range(_K)]
    for e in range(_NE):
        ptr = [pt_v[ch * 16 + e] for ch in range(_T)]
        for k in range(_K):
            ae = _bc(a[k], e)
            for ch in range(_T):
                us[k][ch] = us[k][ch] + ae * ptr[ch]

    wch = [[None] * _T for _ in range(_K)]
    for k in range(_K):
        for ch in range(_T):
            wch[k][ch] = c_v[k, pl.ds(ch * 16, 16)] * (us[k][ch] + beta[k])
    tparts = [[] for _ in range(_K)]
    for ch in range(_T):
        prods = [[] for _ in range(_K)]
        for pp in range(16):
            prow = p_v[ch * 16 + pp]
            for k in range(_K):
                prods[k].append(_bc(wch[k][ch], pp) * prow)
        for k in range(_K):
            tparts[k].append(_tree(prods[k]))
    tv = [None] * _K
    sw = [None] * _K
    for k in range(_K):
        tv[k] = _tree(tparts[k])
        wtot = wch[k][0]
        for ch in range(1, _T):
            wtot = wtot + wch[k][ch]
        sw[k] = _allsum(wtot)

    for k in range(_K):
        o0 = sw[k] * bv_v[jj[k], 0:16]
        o1 = sw[k] * bv_v[jj[k], 16:32]
        for e in range(_NE):
            te = _bc(tv[k], e)
            o0 = o0 + te * wvt_v[k, e]
            o1 = o1 + te * wvt_v[k, 16 + e]
        out_v[k, 0:16] = o0
        out_v[k, 16:32] = o1

    pltpu.sync_copy(out_v, out_hbm.at[wid])


def _make_sc_call():
    return pl.kernel(
        _sc_body,
        out_type=jax.ShapeDtypeStruct((32, _K, _NN), jnp.float32),
        mesh=plsc.VectorSubcoreMesh(core_axis_name="c", subcore_axis_name="s",
                                    num_cores=2, num_subcores=16),
        scratch_types=[
            pltpu.VMEM((_P96, _NE), jnp.float32),
            pltpu.VMEM((_NN, _NE), jnp.float32),
            pltpu.VMEM((_K, _NN, _NE), jnp.float32),
            pltpu.VMEM((_NP, _NN), jnp.float32),
            pltpu.VMEM((_K, _NN, _NE), jnp.float32),
            pltpu.VMEM((_NP, _NN), jnp.float32),
            pltpu.VMEM((_K, _P96), jnp.float32),
            pltpu.VMEM((_P96, _NE), jnp.float32),
            pltpu.VMEM((_P96, _NE), jnp.float32),
            pltpu.VMEM((_NN, _NE), jnp.float32),
            pltpu.VMEM((_K, _NN, _NE), jnp.float32),
            pltpu.VMEM((_K, _NN), jnp.float32),
            pltpu.SemaphoreType.DMA,
            pltpu.SemaphoreType.DMA,
        ],
    )


def kernel(x, Wq, Wk, bk, Wv, bv):
    out = _make_sc_call()(
        x.reshape(_P96, _NE),
        Wq,
        Wk,
        bk,
        Wv,
        bv,
        jnp.asarray(_C_PAIR.reshape(32, _K, _P96)),
    )
    return out.reshape(_T, _NP, _NN)[None]
